# baseline (speedup 1.0000x reference)
import jax
import jax.numpy as jnp
from jax import lax
from jax.experimental import pallas as pl
from jax.experimental.pallas import tpu as pltpu
from jax.experimental.pallas import tpu_sc as plsc

DIM = 64
BATCH = 4096
SEQ = 200
LANES = 16
NUM_CORES = 2
NUM_SUBCORES = 16
NW = NUM_CORES * NUM_SUBCORES
BBLK = 128
NBB = BATCH // BBLK
NBLOCKS = SEQ * NBB
BLK_PER_W = NBLOCKS // NW
SCALE = 8.0


def _embed_body(idx_hbm, table_hbm, pos_hbm, out_hbm,
                idx0, idx1, rows0, rows1, out0, out1, pos_v,
                sg0, sg1, so0, so1):
    wid = lax.axis_index("s") * NUM_CORES + lax.axis_index("c")
    base = wid * BLK_PER_W

    pltpu.sync_copy(pos_hbm, pos_v)

    idx_b = (idx0, idx1)
    rows_b = (rows0, rows1)
    out_b = (out0, out1)
    sg_b = (sg0, sg1)
    so_b = (so0, so1)

    row16 = [lax.iota(jnp.int32, LANES) + (k * LANES) for k in range(8)]

    def stage_in(t, buf):
        g = base + t
        s = g // NBB
        tb = g - s * NBB
        pltpu.sync_copy(idx_hbm.at[s, pl.ds(tb * BBLK, BBLK)], idx_b[buf])
        pltpu.async_copy(table_hbm.at[idx_b[buf]], rows_b[buf], sg_b[buf])

    stage_in(0, 0)
    stage_in(1, 1)

    def step(t, buf, tt):
        g = base + t
        s = g // NBB
        tb = g - s * NBB
        @pl.when(tt >= 1)
        def _():
            pltpu.make_async_copy(
                out_b[buf], out_hbm.at[s, :, tb], so_b[buf]).wait()
        pltpu.make_async_copy(
            table_hbm.at[idx_b[buf]], rows_b[buf], sg_b[buf]).wait()

        rows = rows_b[buf]
        out_v = out_b[buf]

        svec = jnp.full((LANES,), s, jnp.int32)

        def td_body(td, carry):
            for d8 in range(8):
                d = td * 8 + d8
                dvec = jnp.full((LANES,), d, jnp.int32)
                p = plsc.load_gather(pos_v, [svec, dvec])
                for k in range(8):
                    r = plsc.load_gather(rows, [row16[k], dvec])
                    out_v[td, d8, pl.ds(k * LANES, LANES)] = r * SCALE + p
            return carry

        lax.fori_loop(0, 8, td_body, 0)

        @pl.when(t + 2 < BLK_PER_W)
        def _():
            stage_in(t + 2, buf)

        pltpu.async_copy(out_v, out_hbm.at[s, :, tb], so_b[buf])

    def outer(tt, carry):
        step(2 * tt, 0, tt)
        step(2 * tt + 1, 1, tt)
        return carry

    lax.fori_loop(0, BLK_PER_W // 2, outer, 0)

    for t in (BLK_PER_W - 2, BLK_PER_W - 1):
        g = base + t
        s = g // NBB
        tb = g - s * NBB
        buf = t % 2
        pltpu.make_async_copy(out_b[buf], out_hbm.at[s, :, tb], so_b[buf]).wait()


def kernel(inputs, table, pos_enc):
    idx_t = inputs.T.astype(jnp.int32)
    pos = pos_enc.reshape(SEQ, DIM).astype(jnp.float32)
    mesh = plsc.VectorSubcoreMesh(core_axis_name="c", subcore_axis_name="s")
    f = pl.kernel(
        _embed_body,
        out_type=jax.ShapeDtypeStruct((SEQ, 8, NBB, 8, BBLK), jnp.float32),
        mesh=mesh,
        scratch_types=[
            pltpu.VMEM((BBLK,), jnp.int32),
            pltpu.VMEM((BBLK,), jnp.int32),
            pltpu.VMEM((BBLK, DIM), jnp.float32),
            pltpu.VMEM((BBLK, DIM), jnp.float32),
            pltpu.VMEM((8, 8, BBLK), jnp.float32),
            pltpu.VMEM((8, 8, BBLK), jnp.float32),
            pltpu.VMEM((SEQ, DIM), jnp.float32),
            pltpu.SemaphoreType.DMA,
            pltpu.SemaphoreType.DMA,
            pltpu.SemaphoreType.DMA,
            pltpu.SemaphoreType.DMA,
        ],
        compiler_params=pltpu.CompilerParams(
            use_tc_tiling_on_sc=False, needs_layout_passes=False),
    )
    out5 = f(idx_t, table, pos)
    return out5.transpose(2, 4, 0, 1, 3).reshape(BATCH, SEQ, DIM)


# speedup vs baseline: 1.6592x; 1.6592x over previous
import jax
import jax.numpy as jnp
from jax import lax
from jax.experimental import pallas as pl
from jax.experimental.pallas import tpu as pltpu
from jax.experimental.pallas import tpu_sc as plsc

DIM = 64
BATCH = 4096
SEQ = 200
LANES = 16
NUM_CORES = 2
NUM_SUBCORES = 16
NW = NUM_CORES * NUM_SUBCORES
BBLK = 128
NBB = BATCH // BBLK
NBLOCKS = SEQ * NBB
BLK_PER_W = NBLOCKS // NW
SCALE = 8.0


def _embed_body(idx_hbm, table_hbm, pos_hbm, out_hbm,
                idx0, idx1, rows0, rows1, out0, out1, pos_v,
                sg0, sg1, so0, so1):
    wid = lax.axis_index("s") * NUM_CORES + lax.axis_index("c")
    base = wid * BLK_PER_W

    pltpu.sync_copy(pos_hbm, pos_v)

    idx_b = (idx0, idx1)
    rows_b = (rows0, rows1)
    out_b = (out0, out1)
    sg_b = (sg0, sg1)
    so_b = (so0, so1)

    row16 = [lax.iota(jnp.int32, LANES) + (k * LANES) for k in range(8)]

    def stage_in(t, buf):
        g = base + t
        s = g // NBB
        tb = g - s * NBB
        pltpu.sync_copy(idx_hbm.at[s, pl.ds(tb * BBLK, BBLK)], idx_b[buf])
        pltpu.async_copy(table_hbm.at[idx_b[buf]], rows_b[buf], sg_b[buf])

    stage_in(0, 0)
    stage_in(1, 1)

    def step(t, buf, tt):
        g = base + t
        s = g // NBB
        tb = g - s * NBB
        @pl.when(tt >= 1)
        def _():
            pltpu.make_async_copy(
                out_b[buf], out_hbm.at[s, :, tb], so_b[buf]).wait()
        pltpu.make_async_copy(
            table_hbm.at[idx_b[buf]], rows_b[buf], sg_b[buf]).wait()

        rows = rows_b[buf]
        out_v = out_b[buf]

        svec = jnp.full((LANES,), s, jnp.int32)
        iota = row16[0]

        def j_body(j, carry):
            perm = lax.rem(iota + j, jnp.int32(16))
            tdp = lax.div(perm, jnp.int32(8))
            d8p = lax.rem(perm, jnp.int32(8))
            for c in range(4):
                colv = perm + (16 * c)
                pvec = plsc.load_gather(pos_v, [svec, colv])
                tdv = tdp + (2 * c)
                for k in range(8):
                    r = plsc.load_gather(rows, [row16[k], colv])
                    plsc.store_scatter(out_v, [tdv, d8p, row16[k]],
                                       r * SCALE + pvec)
            return carry

        lax.fori_loop(0, LANES, j_body, 0)

        @pl.when(t + 2 < BLK_PER_W)
        def _():
            stage_in(t + 2, buf)

        pltpu.async_copy(out_v, out_hbm.at[s, :, tb], so_b[buf])

    def outer(tt, carry):
        step(2 * tt, 0, tt)
        step(2 * tt + 1, 1, tt)
        return carry

    lax.fori_loop(0, BLK_PER_W // 2, outer, 0)

    for t in (BLK_PER_W - 2, BLK_PER_W - 1):
        g = base + t
        s = g // NBB
        tb = g - s * NBB
        buf = t % 2
        pltpu.make_async_copy(out_b[buf], out_hbm.at[s, :, tb], so_b[buf]).wait()


def kernel(inputs, table, pos_enc):
    idx_t = inputs.T.astype(jnp.int32)
    pos = pos_enc.reshape(SEQ, DIM).astype(jnp.float32)
    mesh = plsc.VectorSubcoreMesh(core_axis_name="c", subcore_axis_name="s")
    f = pl.kernel(
        _embed_body,
        out_type=jax.ShapeDtypeStruct((SEQ, 8, NBB, 8, BBLK), jnp.float32),
        mesh=mesh,
        scratch_types=[
            pltpu.VMEM((BBLK,), jnp.int32),
            pltpu.VMEM((BBLK,), jnp.int32),
            pltpu.VMEM((BBLK, DIM), jnp.float32),
            pltpu.VMEM((BBLK, DIM), jnp.float32),
            pltpu.VMEM((8, 8, BBLK), jnp.float32),
            pltpu.VMEM((8, 8, BBLK), jnp.float32),
            pltpu.VMEM((SEQ, DIM), jnp.float32),
            pltpu.SemaphoreType.DMA,
            pltpu.SemaphoreType.DMA,
            pltpu.SemaphoreType.DMA,
            pltpu.SemaphoreType.DMA,
        ],
        compiler_params=pltpu.CompilerParams(
            use_tc_tiling_on_sc=False, needs_layout_passes=False),
    )
    out5 = f(idx_t, table, pos)
    return out5.transpose(2, 4, 0, 1, 3).reshape(BATCH, SEQ, DIM)


# speedup vs baseline: 1.7869x; 1.0770x over previous
import jax
import jax.numpy as jnp
from jax import lax
from jax.experimental import pallas as pl
from jax.experimental.pallas import tpu as pltpu
from jax.experimental.pallas import tpu_sc as plsc

DIM = 64
BATCH = 4096
SEQ = 200
LANES = 16
NUM_CORES = 2
NUM_SUBCORES = 16
NW = NUM_CORES * NUM_SUBCORES
BBLK = 128
NBB = BATCH // BBLK
NBLOCKS = SEQ * NBB
BLK_PER_W = NBLOCKS // NW
IDX_PER_W = BLK_PER_W * BBLK
SCALE = 8.0
NRB = 4


def _embed_body(idx_hbm, table_hbm, pos_hbm, out_hbm,
                idx_all, rows0, rows1, rows2, rows3, out0, out1, pos_v,
                sg0, sg1, sg2, sg3, so0, so1):
    wid = lax.axis_index("s") * NUM_CORES + lax.axis_index("c")
    base = wid * BLK_PER_W

    rows_b = (rows0, rows1, rows2, rows3)
    out_b = (out0, out1)
    sg_b = (sg0, sg1, sg2, sg3)
    so_b = (so0, so1)

    pltpu.sync_copy(idx_hbm.at[pl.ds(wid * IDX_PER_W, IDX_PER_W)], idx_all)
    pltpu.sync_copy(pos_hbm, pos_v)

    iota = lax.iota(jnp.int32, LANES)
    row16 = [iota + (k * LANES) for k in range(8)]

    def block_coords(t):
        h = base + t
        st = h // 256
        rem = h - st * 256
        tb = rem // 8
        s8 = rem - tb * 8
        return st * 8 + s8, tb

    def start_gather(t, rbuf):
        pltpu.async_copy(
            table_hbm.at[idx_all.at[pl.ds(t * BBLK, BBLK)]],
            rows_b[rbuf], sg_b[rbuf])

    for r in range(NRB):
        start_gather(r, r)

    def step(t, rbuf, obuf):
        s, tb = block_coords(t)
        @pl.when(t >= 2)
        def _():
            s2, tb2 = block_coords(t - 2)
            pltpu.make_async_copy(
                out_b[obuf], out_hbm.at[s2, :, tb2], so_b[obuf]).wait()
        pltpu.make_async_copy(
            table_hbm.at[idx_all.at[pl.ds(t * BBLK, BBLK)]],
            rows_b[rbuf], sg_b[rbuf]).wait()

        rows = rows_b[rbuf]
        out_v = out_b[obuf]
        svec = jnp.full((LANES,), s, jnp.int32)

        def j_body(j, carry):
            perm = lax.rem(iota + j, jnp.int32(16))
            tdp = lax.div(perm, jnp.int32(8))
            d8p = perm - tdp * 8
            inner = [d8p * 128 + row16[k] for k in range(8)]
            for c in range(4):
                colv = perm + (16 * c)
                pvec = plsc.load_gather(pos_v, [svec, colv])
                tdv = tdp + (2 * c)
                for k in range(8):
                    r = plsc.load_gather(rows, [row16[k], colv])
                    plsc.store_scatter(out_v, [tdv, inner[k]],
                                       r * SCALE + pvec)
            return carry

        lax.fori_loop(0, LANES, j_body, 0)

        @pl.when(t + NRB < BLK_PER_W)
        def _():
            start_gather(t + NRB, rbuf)

        pltpu.async_copy(out_v, out_hbm.at[s, :, tb], so_b[obuf])

    def outer(tt, carry):
        t0 = tt * NRB
        for r in range(NRB):
            step(t0 + r, r, r % 2)
        return carry

    lax.fori_loop(0, BLK_PER_W // NRB, outer, 0)

    for t in (BLK_PER_W - 2, BLK_PER_W - 1):
        s, tb = block_coords(t)
        pltpu.make_async_copy(out_b[t % 2], out_hbm.at[s, :, tb],
                              so_b[t % 2]).wait()


def kernel(inputs, table, pos_enc):
    idx_flat = (inputs.astype(jnp.int32)
                .reshape(NBB, BBLK, SEQ // 8, 8)
                .transpose(2, 0, 3, 1)
                .reshape(NBLOCKS * BBLK))
    pos = pos_enc.reshape(SEQ, DIM).astype(jnp.float32)
    mesh = plsc.VectorSubcoreMesh(core_axis_name="c", subcore_axis_name="s")
    f = pl.kernel(
        _embed_body,
        out_type=jax.ShapeDtypeStruct((SEQ, 8, NBB, 1024), jnp.float32),
        mesh=mesh,
        scratch_types=[
            pltpu.VMEM((IDX_PER_W,), jnp.int32),
            pltpu.VMEM((BBLK, DIM), jnp.float32),
            pltpu.VMEM((BBLK, DIM), jnp.float32),
            pltpu.VMEM((BBLK, DIM), jnp.float32),
            pltpu.VMEM((BBLK, DIM), jnp.float32),
            pltpu.VMEM((8, 8 * BBLK), jnp.float32),
            pltpu.VMEM((8, 8 * BBLK), jnp.float32),
            pltpu.VMEM((SEQ, DIM), jnp.float32),
            pltpu.SemaphoreType.DMA,
            pltpu.SemaphoreType.DMA,
            pltpu.SemaphoreType.DMA,
            pltpu.SemaphoreType.DMA,
            pltpu.SemaphoreType.DMA,
            pltpu.SemaphoreType.DMA,
        ],
        compiler_params=pltpu.CompilerParams(
            use_tc_tiling_on_sc=False, needs_layout_passes=False),
    )
    out5 = f(idx_flat, table, pos)
    return (out5.reshape(SEQ, 8, NBB, 8, BBLK)
            .transpose(2, 4, 0, 1, 3)
            .reshape(BATCH, SEQ, DIM))
